# Initial kernel scaffold; baseline (speedup 1.0000x reference)
import jax
import jax.numpy as jnp
from jax.experimental import pallas as pl


def kernel(table, tokens):
    raise NotImplementedError("write your pallas kernel here")


# speedup vs baseline: 3.5899x; 3.5899x over previous
import jax
import jax.numpy as jnp
from jax import lax
from jax.experimental import pallas as pl
from jax.experimental.pallas import tpu as pltpu
from jax.experimental.pallas import tpu_sc as plsc

D = 64
B = 4096 * 200
NC, NS = 2, 16
NW = NC * NS
ROWS_PER_GATHER = 128
B_PER_W = B // NW
SLABS_PER_W = B_PER_W // ROWS_PER_GATHER
GATHERS_PER_CHUNK = 8
CHUNK = ROWS_PER_GATHER * GATHERS_PER_CHUNK
N_CHUNKS = B_PER_W // CHUNK


def _emb_body(table_hbm, tok_hbm, out_hbm, idx_v, rows_v, sem):
    wid = lax.axis_index("s") * NC + lax.axis_index("c")
    pltpu.sync_copy(tok_hbm.at[pl.ds(wid * SLABS_PER_W, SLABS_PER_W)], idx_v)

    def chunk_body(i, carry):
        handles = []
        for j in range(GATHERS_PER_CHUNK):
            h = pltpu.async_copy(
                table_hbm.at[idx_v.at[i * GATHERS_PER_CHUNK + j]],
                rows_v.at[pl.ds(j * ROWS_PER_GATHER, ROWS_PER_GATHER)],
                sem,
            )
            handles.append(h)
        for h in handles:
            h.wait()
        pltpu.sync_copy(
            rows_v, out_hbm.at[pl.ds(wid * B_PER_W + i * CHUNK, CHUNK)]
        )
        return carry

    lax.fori_loop(0, N_CHUNKS, chunk_body, 0)


def kernel(table, tokens):
    tok2d = tokens.reshape(B // ROWS_PER_GATHER, ROWS_PER_GATHER)
    mesh = plsc.VectorSubcoreMesh(core_axis_name="c", subcore_axis_name="s")
    out = pl.kernel(
        _emb_body,
        mesh=mesh,
        compiler_params=pltpu.CompilerParams(use_tc_tiling_on_sc=False),
        out_type=jax.ShapeDtypeStruct((B, D), jnp.float32),
        scratch_types=[
            pltpu.VMEM((SLABS_PER_W, ROWS_PER_GATHER), jnp.int32),
            pltpu.VMEM((CHUNK, D), jnp.float32),
            pltpu.SemaphoreType.DMA,
        ],
    )(table, tok2d)
    return out.reshape(tokens.shape[0], tokens.shape[1], D)
